# baseline (speedup 1.0000x reference)
import functools

import jax
import jax.numpy as jnp
from jax import lax
from jax.experimental import pallas as pl
from jax.experimental.pallas import tpu as pltpu
from jax.experimental.pallas import tpu_sc as plsc

N = 10000
E = 320000
IN_DIM = 128
HID = 256
NEXP = 8
NGRAPH = 64

NC = 2
NS = 16
NW = NC * NS
OWN = 320
NPAD = NW * OWN
ACCR = OWN + 16
CH = 1000
NCHUNK = E // CH
G = 80

BB = 1000
NB = N // BB


@functools.lru_cache(maxsize=1)
def _mesh():
    return plsc.VectorSubcoreMesh(
        core_axis_name="c", subcore_axis_name="s",
        num_cores=NC, num_subcores=NS)


def _enc_body(x_ref, bcol_ref, wenc_ref, benc_ref, w1_ref, b1_ref, w2_ref,
              b2_ref, h_ref, sem_ref, nc_ref):
    i = pl.program_id(0)
    x = x_ref[...]
    h = jnp.maximum(jnp.dot(x, wenc_ref[...],
                            preferred_element_type=jnp.float32)
                    + benc_ref[...], 0.0)
    h_ref[...] = h
    hr = jnp.maximum(jnp.dot(h, w1_ref[...],
                             preferred_element_type=jnp.float32)
                     + b1_ref[...], 0.0)
    sem_ref[...] = jnp.dot(hr, w2_ref[...],
                           preferred_element_type=jnp.float32) + b2_ref[...]
    b = bcol_ref[...]
    oh = (b == lax.broadcasted_iota(jnp.int32, (BB, NGRAPH), 1)
          ).astype(jnp.float32)
    cnt = jnp.sum(oh, axis=0, keepdims=True)

    @pl.when(i == 0)
    def _():
        nc_ref[...] = jnp.zeros_like(nc_ref)

    nc_ref[...] += jnp.broadcast_to(cnt, (8, NGRAPH))


def _phase_a(x, bcol, W_enc, b_enc, sem_W1, sem_b1, sem_W2, sem_b2):
    full = lambda s: pl.BlockSpec(s, lambda i: tuple(0 for _ in s))
    return pl.pallas_call(
        _enc_body,
        grid=(NB,),
        in_specs=[
            pl.BlockSpec((BB, IN_DIM), lambda i: (i, 0)),
            pl.BlockSpec((BB, 1), lambda i: (i, 0)),
            full((IN_DIM, HID)), full((1, HID)),
            full((HID, HID)), full((1, HID)),
            full((HID, NEXP)), full((1, NEXP)),
        ],
        out_specs=[
            pl.BlockSpec((BB, HID), lambda i: (i, 0)),
            pl.BlockSpec((BB, NEXP), lambda i: (i, 0)),
            pl.BlockSpec((8, NGRAPH), lambda i: (0, 0)),
        ],
        out_shape=[
            jax.ShapeDtypeStruct((N, HID), jnp.float32),
            jax.ShapeDtypeStruct((N, NEXP), jnp.float32),
            jax.ShapeDtypeStruct((8, NGRAPH), jnp.float32),
        ],
    )(x, bcol, W_enc, b_enc.reshape(1, HID), sem_W1, sem_b1.reshape(1, HID),
      sem_W2, sem_b2.reshape(1, NEXP))


def _zero_acc(acc_v):
    zf = jnp.zeros((16,), jnp.float32)

    def _z(r, _):
        for c in range(HID // 16):
            acc_v[r, pl.ds(c * 16, 16)] = zf
        return 0

    lax.fori_loop(0, ACCR, _z, 0)


def _copy_out(acc_v, out_ref, wid):
    for i in range(OWN // G):
        off = pl.multiple_of(wid * OWN + i * G, 8)
        pltpu.sync_copy(acc_v.at[pl.ds(i * G, G)], out_ref.at[pl.ds(off, G)])


def _phase_b_body(h_hbm, src_hbm, dst_hbm, batch_hbm,
                  agg_out, dcnt_out, ecnt_out,
                  batch_v, srcv, dstv, cgi, cdst, rows_v, acc_v, dcnt_t,
                  ecnt_t, sem):
    cid = lax.axis_index("c")
    tid = lax.axis_index("s")
    wid = cid * NS + tid
    base = wid * OWN
    io16 = lax.iota(jnp.int32, 16)
    zf = jnp.zeros((16,), jnp.float32)
    one16 = jnp.full((16,), 1.0, jnp.float32)

    _zero_acc(acc_v)

    def _zc(r, _):
        dcnt_t[pl.ds(r * 16, 16)] = zf
        return 0

    lax.fori_loop(0, ACCR // 16, _zc, 0)
    for r in range(8):
        ecnt_t[pl.ds(r * 16, 16)] = zf

    pltpu.sync_copy(batch_hbm, batch_v)

    def _chunk(ci, _):
        off = pl.multiple_of(ci * CH, 8)
        pltpu.sync_copy(src_hbm.at[pl.ds(off, CH)], srcv)
        pltpu.sync_copy(dst_hbm.at[pl.ds(off, CH)], dstv)

        def _pf(r, _):
            cgi[pl.ds(r * 16, 16)] = io16 * 0
            cdst[pl.ds(r * 16, 16)] = OWN + (io16 & 15)
            return 0

        lax.fori_loop(0, CH // 16, _pf, 0)

        lane15 = jnp.full((16,), 15, jnp.int32)

        def _scan(j, mv):
            s = srcv[pl.ds(j * 16, 16)]
            d = dstv[pl.ds(j * 16, 16)]
            dl = d - base
            msk = (dl >= 0) & (dl < OWN)
            csum = plsc.cumsum(msk.astype(jnp.int32))
            pos = mv + csum - 1
            plsc.store_scatter(cgi, [pos], s, mask=msk)
            plsc.store_scatter(cdst, [pos], dl, mask=msk)
            return mv + jnp.take(csum, lane15)

        mv = lax.fori_loop(0, CH // 16, _scan,
                           jnp.zeros((16,), jnp.int32))
        m = mv[0]
        ng = (m + (G - 1)) // G

        def _fire(g, _):
            go = pl.multiple_of(g * G, 8)
            pltpu.async_copy(h_hbm.at[cgi.at[pl.ds(go, G)]], rows_v,
                             sem).wait()

            def _addgrp(rg, _):
                lbase = go + rg * 16
                slot16 = cdst[pl.ds(lbase, 16)]
                live = (lbase + io16) < m
                plsc.addupdate_scatter(dcnt_t, [slot16], one16, mask=live)
                s16 = cgi[pl.ds(lbase, 16)]
                bg = plsc.load_gather(batch_v, [s16])
                plsc.addupdate_scatter(ecnt_t, [bg], one16, mask=live)
                for r in range(16):
                    slot = slot16[r]
                    for c in range(HID // 16):
                        sl = pl.ds(c * 16, 16)
                        plsc.addupdate(acc_v.at[slot, sl],
                                       rows_v[rg * 16 + r, sl])
                return 0

            lax.fori_loop(0, G // 16, _addgrp, 0)
            return 0

        lax.fori_loop(0, ng, _fire, 0)
        return 0

    lax.fori_loop(0, NCHUNK, _chunk, 0)

    _copy_out(acc_v, agg_out, wid)
    off = pl.multiple_of(wid * OWN, 8)
    pltpu.sync_copy(dcnt_t.at[pl.ds(0, OWN)], dcnt_out.at[pl.ds(off, OWN)])
    pltpu.sync_copy(ecnt_t, ecnt_out.at[wid])


@functools.lru_cache(maxsize=1)
def _phase_b():
    return pl.kernel(
        _phase_b_body, mesh=_mesh(),
        compiler_params=pltpu.CompilerParams(needs_layout_passes=False),
        out_type=[
            jax.ShapeDtypeStruct((NPAD, HID), jnp.float32),
            jax.ShapeDtypeStruct((NPAD,), jnp.float32),
            jax.ShapeDtypeStruct((NW, 128), jnp.float32),
        ],
        scratch_types=[
            pltpu.VMEM((N,), jnp.int32),
            pltpu.VMEM((CH,), jnp.int32),
            pltpu.VMEM((CH,), jnp.int32),
            pltpu.VMEM((CH + 16,), jnp.int32),
            pltpu.VMEM((CH + 16,), jnp.int32),
            pltpu.VMEM((G, HID), jnp.float32),
            pltpu.VMEM((ACCR, HID), jnp.float32),
            pltpu.VMEM((ACCR,), jnp.float32),
            pltpu.VMEM((128,), jnp.float32),
            pltpu.SemaphoreType.DMA,
        ])


def _mid_body(bcol_ref, sem_ref, nc_ref, ec_ref, h_ref, agg_ref, dc_ref,
              sw1_ref, sb1_ref, sw2_ref, sb2_ref, ew1_ref, eb1_ref,
              he1_ref, sel0_ref, sel1_ref, e0_ref, e1_ref, g0_ref, g1_ref):
    b = bcol_ref[...]
    oh = (b == lax.broadcasted_iota(jnp.int32, (BB, NGRAPH), 1)
          ).astype(jnp.float32)
    lognc = jnp.log(1.0 + nc_ref[...])
    ones_w = jnp.ones((1, NW), jnp.float32)
    ec_sum = lax.dot_general(ones_w, ec_ref[...], (((1,), (0,)), ((), ())),
                             preferred_element_type=jnp.float32)
    logec = jnp.log(1.0 + ec_sum[:, :NGRAPH])
    sz1 = lax.dot_general(oh, lognc, (((1,), (1,)), ((), ())),
                          preferred_element_type=jnp.float32)[:, 0:1]
    sz2 = lax.dot_general(oh, logec, (((1,), (1,)), ((), ())),
                          preferred_element_type=jnp.float32)
    pre = (jnp.dot(sz1, sw1_ref[0:1, :], preferred_element_type=jnp.float32)
           + jnp.dot(sz2, sw1_ref[1:2, :], preferred_element_type=jnp.float32)
           + sb1_ref[...])
    hs = jnp.maximum(pre, 0.0)
    str_logits = jnp.dot(hs, sw2_ref[...],
                         preferred_element_type=jnp.float32) + sb2_ref[...]
    logits = 0.5 * sem_ref[...] + 0.5 * str_logits

    io = lax.broadcasted_iota(jnp.int32, (BB, NEXP), 1)
    v1 = jnp.max(logits, axis=1, keepdims=True)
    e0 = jnp.min(jnp.where(logits == v1, io, NEXP), axis=1, keepdims=True)
    l2 = jnp.where(io == e0, jnp.float32(-1e30), logits)
    v2 = jnp.max(l2, axis=1, keepdims=True)
    e1 = jnp.min(jnp.where(l2 == v2, io, NEXP), axis=1, keepdims=True)
    sexp = jnp.exp(v2 - v1)
    inv = 1.0 / (1.0 + sexp)
    g0_ref[...] = inv
    g1_ref[...] = sexp * inv
    e0_ref[...] = e0
    e1_ref[...] = e1

    invdeg = 1.0 / jnp.maximum(dc_ref[...], 1.0)
    z = agg_ref[...] * invdeg + h_ref[...]
    sel0 = jnp.zeros((BB, HID), jnp.float32)
    sel1 = jnp.zeros((BB, HID), jnp.float32)
    for e in range(NEXP):
        he = jnp.maximum(
            jnp.dot(z, ew1_ref[e], preferred_element_type=jnp.float32)
            + eb1_ref[e:e + 1, :], 0.0)
        he1_ref[e, :, :] = he
        sel0 += (e0 == e).astype(jnp.float32) * he
        sel1 += (e1 == e).astype(jnp.float32) * he
    sel0_ref[...] = sel0
    sel1_ref[...] = sel1


def _phase_c(bcol, sem, nc, ec32, h, agg0, dcol, str_W1, str_b1,
             str_W2, str_b2, exp_W1, exp_b1):
    full = lambda s: pl.BlockSpec(s, lambda i: tuple(0 for _ in s))
    rowf = pl.BlockSpec((BB, HID), lambda i: (i, 0))
    col = pl.BlockSpec((BB, 1), lambda i: (i, 0))
    return pl.pallas_call(
        _mid_body,
        grid=(NB,),
        in_specs=[
            col,
            pl.BlockSpec((BB, NEXP), lambda i: (i, 0)),
            full((8, NGRAPH)), full((NW, 128)),
            rowf, rowf, col,
            full((2, HID)), full((1, HID)), full((HID, NEXP)),
            full((1, NEXP)), full((NEXP, HID, HID)), full((NEXP, HID)),
        ],
        out_specs=[
            pl.BlockSpec((NEXP, BB, HID), lambda i: (0, i, 0)),
            rowf, rowf, col, col, col, col,
        ],
        out_shape=[
            jax.ShapeDtypeStruct((NEXP, N, HID), jnp.float32),
            jax.ShapeDtypeStruct((N, HID), jnp.float32),
            jax.ShapeDtypeStruct((N, HID), jnp.float32),
            jax.ShapeDtypeStruct((N, 1), jnp.int32),
            jax.ShapeDtypeStruct((N, 1), jnp.int32),
            jax.ShapeDtypeStruct((N, 1), jnp.float32),
            jax.ShapeDtypeStruct((N, 1), jnp.float32),
        ],
    )(bcol, sem, nc, ec32, h, agg0, dcol, str_W1,
      str_b1.reshape(1, HID), str_W2, str_b2.reshape(1, NEXP), exp_W1, exp_b1)


def _phase_d_body(he1_hbm, src_hbm, dst_hbm, ek_hbm, a_out,
                  ek_v, srcv, dstv, cgi, cdst, rows_v, acc_v, sem):
    cid = lax.axis_index("c")
    tid = lax.axis_index("s")
    wid = cid * NS + tid
    base = wid * OWN
    io16 = lax.iota(jnp.int32, 16)

    _zero_acc(acc_v)
    pltpu.sync_copy(ek_hbm, ek_v)

    def _chunk(ci, _):
        off = pl.multiple_of(ci * CH, 8)
        pltpu.sync_copy(src_hbm.at[pl.ds(off, CH)], srcv)
        pltpu.sync_copy(dst_hbm.at[pl.ds(off, CH)], dstv)

        def _pf(r, _):
            cgi[pl.ds(r * 16, 16)] = io16 * 0
            cdst[pl.ds(r * 16, 16)] = OWN + (io16 & 15)
            return 0

        lax.fori_loop(0, CH // 16, _pf, 0)

        lane15 = jnp.full((16,), 15, jnp.int32)

        def _scan(j, mv):
            s = srcv[pl.ds(j * 16, 16)]
            d = dstv[pl.ds(j * 16, 16)]
            dl = d - base
            msk = (dl >= 0) & (dl < OWN)
            csum = plsc.cumsum(msk.astype(jnp.int32))
            pos = mv + csum - 1
            ee = plsc.load_gather(ek_v, [d])
            plsc.store_scatter(cgi, [pos], ee * N + s, mask=msk)
            plsc.store_scatter(cdst, [pos], dl, mask=msk)
            return mv + jnp.take(csum, lane15)

        mv = lax.fori_loop(0, CH // 16, _scan,
                           jnp.zeros((16,), jnp.int32))
        m = mv[0]
        ng = (m + (G - 1)) // G

        def _fire(g, _):
            go = pl.multiple_of(g * G, 8)
            pltpu.async_copy(he1_hbm.at[cgi.at[pl.ds(go, G)]], rows_v,
                             sem).wait()

            def _addgrp(rg, _):
                lbase = go + rg * 16
                slot16 = cdst[pl.ds(lbase, 16)]
                for r in range(16):
                    slot = slot16[r]
                    for c in range(HID // 16):
                        sl = pl.ds(c * 16, 16)
                        plsc.addupdate(acc_v.at[slot, sl],
                                       rows_v[rg * 16 + r, sl])
                return 0

            lax.fori_loop(0, G // 16, _addgrp, 0)
            return 0

        lax.fori_loop(0, ng, _fire, 0)
        return 0

    lax.fori_loop(0, NCHUNK, _chunk, 0)
    _copy_out(acc_v, a_out, wid)


@functools.lru_cache(maxsize=1)
def _phase_d():
    return pl.kernel(
        _phase_d_body, mesh=_mesh(),
        compiler_params=pltpu.CompilerParams(needs_layout_passes=False),
        out_type=jax.ShapeDtypeStruct((NPAD, HID), jnp.float32),
        scratch_types=[
            pltpu.VMEM((N,), jnp.int32),
            pltpu.VMEM((CH,), jnp.int32),
            pltpu.VMEM((CH,), jnp.int32),
            pltpu.VMEM((CH + 16,), jnp.int32),
            pltpu.VMEM((CH + 16,), jnp.int32),
            pltpu.VMEM((G, HID), jnp.float32),
            pltpu.VMEM((ACCR, HID), jnp.float32),
            pltpu.SemaphoreType.DMA,
        ])


def _out_body(a0_ref, a1_ref, sel0_ref, sel1_ref, e0_ref, e1_ref, g0_ref,
              g1_ref, dc_ref, ew2_ref, eb2_ref, out_ref):
    invdeg = 1.0 / jnp.maximum(dc_ref[...], 1.0)
    z0 = a0_ref[...] * invdeg + sel0_ref[...]
    z1 = a1_ref[...] * invdeg + sel1_ref[...]
    e0 = e0_ref[...]
    e1 = e1_ref[...]
    g0 = g0_ref[...]
    g1 = g1_ref[...]
    out = jnp.zeros((BB, HID), jnp.float32)
    for e in range(NEXP):
        y0 = jnp.maximum(
            jnp.dot(z0, ew2_ref[e], preferred_element_type=jnp.float32)
            + eb2_ref[e:e + 1, :], 0.0)
        out += (g0 * (e0 == e).astype(jnp.float32)) * y0
        y1 = jnp.maximum(
            jnp.dot(z1, ew2_ref[e], preferred_element_type=jnp.float32)
            + eb2_ref[e:e + 1, :], 0.0)
        out += (g1 * (e1 == e).astype(jnp.float32)) * y1
    out_ref[...] = out


def _phase_e(a0, a1, sel0, sel1, e0, e1, g0, g1, dcol, exp_W2, exp_b2):
    full = lambda s: pl.BlockSpec(s, lambda i: tuple(0 for _ in s))
    rowf = pl.BlockSpec((BB, HID), lambda i: (i, 0))
    col = pl.BlockSpec((BB, 1), lambda i: (i, 0))
    return pl.pallas_call(
        _out_body,
        grid=(NB,),
        in_specs=[rowf, rowf, rowf, rowf, col, col, col, col, col,
                  full((NEXP, HID, HID)), full((NEXP, HID))],
        out_specs=rowf,
        out_shape=jax.ShapeDtypeStruct((N, HID), jnp.float32),
    )(a0, a1, sel0, sel1, e0, e1, g0, g1, dcol, exp_W2, exp_b2)


def kernel(x, edge_index, batch, W_enc, b_enc, sem_W1, sem_b1, sem_W2, sem_b2,
           str_W1, str_b1, str_W2, str_b2, exp_W1, exp_b1, exp_W2, exp_b2):
    src = edge_index[0]
    dst = edge_index[1]
    bcol = batch.reshape(N, 1)

    h, sem, nc = _phase_a(x, bcol, W_enc, b_enc, sem_W1, sem_b1, sem_W2,
                          sem_b2)

    aggB, dcntB, ecntB = _phase_b()(h, src, dst, batch)
    agg0 = aggB[:N]
    dcol = dcntB[:N, None]

    he1, sel0, sel1, e0c, e1c, g0c, g1c = _phase_c(
        bcol, sem, nc, ecntB, h, agg0, dcol, str_W1, str_b1,
        str_W2, str_b2, exp_W1, exp_b1)

    he1_sc = he1.reshape(NEXP * N, HID)
    a0B = _phase_d()(he1_sc, src, dst, e0c.reshape(N))
    a1B = _phase_d()(he1_sc, src, dst, e1c.reshape(N))

    return _phase_e(a0B[:N], a1B[:N], sel0, sel1, e0c, e1c, g0c, g1c, dcol,
                    exp_W2, exp_b2)


# speedup vs baseline: 5.9794x; 5.9794x over previous
import functools

import jax
import jax.numpy as jnp
from jax import lax
from jax.experimental import pallas as pl
from jax.experimental.pallas import tpu as pltpu
from jax.experimental.pallas import tpu_sc as plsc

N = 10000
E = 320000
IN_DIM = 128
HID = 256
NEXP = 8
NGRAPH = 64

NC = 2
NS = 16
NW = NC * NS
OWN = 320
NPAD = NW * OWN
ACCR = OWN + 16
CH = 4000
NCHUNK = E // CH
G = 64

BB = 1000
NB = N // BB


@functools.lru_cache(maxsize=1)
def _mesh():
    return plsc.VectorSubcoreMesh(
        core_axis_name="c", subcore_axis_name="s",
        num_cores=NC, num_subcores=NS)


def _enc_body(x_ref, bcol_ref, wenc_ref, benc_ref, w1_ref, b1_ref, w2_ref,
              b2_ref, h_ref, sem_ref, nc_ref):
    i = pl.program_id(0)
    x = x_ref[...]
    h = jnp.maximum(jnp.dot(x, wenc_ref[...],
                            preferred_element_type=jnp.float32)
                    + benc_ref[...], 0.0)
    h_ref[...] = h
    hr = jnp.maximum(jnp.dot(h, w1_ref[...],
                             preferred_element_type=jnp.float32)
                     + b1_ref[...], 0.0)
    sem_ref[...] = jnp.dot(hr, w2_ref[...],
                           preferred_element_type=jnp.float32) + b2_ref[...]
    b = bcol_ref[...]
    oh = (b == lax.broadcasted_iota(jnp.int32, (BB, NGRAPH), 1)
          ).astype(jnp.float32)
    cnt = jnp.sum(oh, axis=0, keepdims=True)

    @pl.when(i == 0)
    def _():
        nc_ref[...] = jnp.zeros_like(nc_ref)

    nc_ref[...] += jnp.broadcast_to(cnt, (8, NGRAPH))


def _phase_a(x, bcol, W_enc, b_enc, sem_W1, sem_b1, sem_W2, sem_b2):
    full = lambda s: pl.BlockSpec(s, lambda i: tuple(0 for _ in s))
    return pl.pallas_call(
        _enc_body,
        grid=(NB,),
        in_specs=[
            pl.BlockSpec((BB, IN_DIM), lambda i: (i, 0)),
            pl.BlockSpec((BB, 1), lambda i: (i, 0)),
            full((IN_DIM, HID)), full((1, HID)),
            full((HID, HID)), full((1, HID)),
            full((HID, NEXP)), full((1, NEXP)),
        ],
        out_specs=[
            pl.BlockSpec((BB, HID), lambda i: (i, 0)),
            pl.BlockSpec((BB, NEXP), lambda i: (i, 0)),
            pl.BlockSpec((8, NGRAPH), lambda i: (0, 0)),
        ],
        out_shape=[
            jax.ShapeDtypeStruct((N, HID), jnp.float32),
            jax.ShapeDtypeStruct((N, NEXP), jnp.float32),
            jax.ShapeDtypeStruct((8, NGRAPH), jnp.float32),
        ],
    )(x, bcol, W_enc, b_enc.reshape(1, HID), sem_W1, sem_b1.reshape(1, HID),
      sem_W2, sem_b2.reshape(1, NEXP))


def _zero_acc(acc_v):
    zf = jnp.zeros((16,), jnp.float32)

    def _z(r, _):
        for c in range(HID // 16):
            acc_v[r, pl.ds(c * 16, 16)] = zf
        return 0

    lax.fori_loop(0, ACCR, _z, 0)


def _copy_out(acc_v, out_ref, wid):
    for i in range(OWN // G):
        off = pl.multiple_of(wid * OWN + i * G, 8)
        pltpu.sync_copy(acc_v.at[pl.ds(i * G, G)], out_ref.at[pl.ds(off, G)])


def _phase_b_body(h_hbm, src_hbm, dst_hbm, batch_hbm,
                  agg_out, dcnt_out, ecnt_out,
                  batch_v, srcv, dstv, cgi, cdst, rows_v, acc_v, dcnt_t,
                  ecnt_t, sem):
    cid = lax.axis_index("c")
    tid = lax.axis_index("s")
    wid = cid * NS + tid
    base = wid * OWN
    io16 = lax.iota(jnp.int32, 16)
    zf = jnp.zeros((16,), jnp.float32)
    one16 = jnp.full((16,), 1.0, jnp.float32)

    _zero_acc(acc_v)

    def _zc(r, _):
        dcnt_t[pl.ds(r * 16, 16)] = zf
        return 0

    lax.fori_loop(0, ACCR // 16, _zc, 0)
    for r in range(8):
        ecnt_t[pl.ds(r * 16, 16)] = zf

    pltpu.sync_copy(batch_hbm, batch_v)

    def _chunk(ci, _):
        off = pl.multiple_of(ci * CH, 8)
        pltpu.sync_copy(src_hbm.at[pl.ds(off, CH)], srcv)
        pltpu.sync_copy(dst_hbm.at[pl.ds(off, CH)], dstv)

        def _pf(r, _):
            cgi[pl.ds(r * 16, 16)] = io16 * 0
            cdst[pl.ds(r * 16, 16)] = OWN + (io16 & 15)
            return 0

        lax.fori_loop(0, CH // 16, _pf, 0)

        lane15 = jnp.full((16,), 15, jnp.int32)

        def _scan(j, mv):
            s = srcv[pl.ds(j * 16, 16)]
            d = dstv[pl.ds(j * 16, 16)]
            dl = d - base
            msk = (dl >= 0) & (dl < OWN)
            csum = plsc.cumsum(msk.astype(jnp.int32))
            pos = mv + csum - 1
            plsc.store_scatter(cgi, [pos], s, mask=msk)
            plsc.store_scatter(cdst, [pos], dl, mask=msk)
            return mv + jnp.take(csum, lane15)

        mv = lax.fori_loop(0, CH // 16, _scan,
                           jnp.zeros((16,), jnp.int32))
        m = mv[0]
        ng = (m + (G - 1)) // G

        def _fire(g, _):
            go = pl.multiple_of(g * G, 8)
            pltpu.async_copy(h_hbm.at[cgi.at[pl.ds(go, G)]], rows_v,
                             sem).wait()

            def _addgrp(rg, _):
                lbase = go + rg * 16
                slot16 = cdst[pl.ds(lbase, 16)]
                live = (lbase + io16) < m
                plsc.addupdate_scatter(dcnt_t, [slot16], one16, mask=live)
                s16 = cgi[pl.ds(lbase, 16)]
                bg = plsc.load_gather(batch_v, [s16])
                plsc.addupdate_scatter(ecnt_t, [bg], one16, mask=live)
                for r in range(16):
                    slot = slot16[r]
                    for c in range(HID // 16):
                        sl = pl.ds(c * 16, 16)
                        plsc.addupdate(acc_v.at[slot, sl],
                                       rows_v[rg * 16 + r, sl])
                return 0

            lax.fori_loop(0, G // 16, _addgrp, 0)
            return 0

        lax.fori_loop(0, ng, _fire, 0)
        return 0

    lax.fori_loop(0, NCHUNK, _chunk, 0)

    _copy_out(acc_v, agg_out, wid)
    off = pl.multiple_of(wid * OWN, 8)
    pltpu.sync_copy(dcnt_t.at[pl.ds(0, OWN)], dcnt_out.at[pl.ds(off, OWN)])
    pltpu.sync_copy(ecnt_t, ecnt_out.at[wid])


@functools.lru_cache(maxsize=1)
def _phase_b():
    return pl.kernel(
        _phase_b_body, mesh=_mesh(),
        compiler_params=pltpu.CompilerParams(needs_layout_passes=False),
        out_type=[
            jax.ShapeDtypeStruct((NPAD, HID), jnp.float32),
            jax.ShapeDtypeStruct((NPAD,), jnp.float32),
            jax.ShapeDtypeStruct((NW, 128), jnp.float32),
        ],
        scratch_types=[
            pltpu.VMEM((N,), jnp.int32),
            pltpu.VMEM((CH,), jnp.int32),
            pltpu.VMEM((CH,), jnp.int32),
            pltpu.VMEM((CH + 16,), jnp.int32),
            pltpu.VMEM((CH + 16,), jnp.int32),
            pltpu.VMEM((G, HID), jnp.float32),
            pltpu.VMEM((ACCR, HID), jnp.float32),
            pltpu.VMEM((ACCR,), jnp.float32),
            pltpu.VMEM((128,), jnp.float32),
            pltpu.SemaphoreType.DMA,
        ])


def _mid_body(bcol_ref, sem_ref, nc_ref, ec_ref, h_ref, agg_ref, dc_ref,
              sw1_ref, sb1_ref, sw2_ref, sb2_ref, ew1_ref, eb1_ref,
              he1_ref, sel0_ref, sel1_ref, e0_ref, e1_ref, g0_ref, g1_ref):
    b = bcol_ref[...]
    oh = (b == lax.broadcasted_iota(jnp.int32, (BB, NGRAPH), 1)
          ).astype(jnp.float32)
    lognc = jnp.log(1.0 + nc_ref[...])
    ones_w = jnp.ones((1, NW), jnp.float32)
    ec_sum = lax.dot_general(ones_w, ec_ref[...], (((1,), (0,)), ((), ())),
                             preferred_element_type=jnp.float32)
    logec = jnp.log(1.0 + ec_sum[:, :NGRAPH])
    sz1 = lax.dot_general(oh, lognc, (((1,), (1,)), ((), ())),
                          preferred_element_type=jnp.float32)[:, 0:1]
    sz2 = lax.dot_general(oh, logec, (((1,), (1,)), ((), ())),
                          preferred_element_type=jnp.float32)
    pre = (jnp.dot(sz1, sw1_ref[0:1, :], preferred_element_type=jnp.float32)
           + jnp.dot(sz2, sw1_ref[1:2, :], preferred_element_type=jnp.float32)
           + sb1_ref[...])
    hs = jnp.maximum(pre, 0.0)
    str_logits = jnp.dot(hs, sw2_ref[...],
                         preferred_element_type=jnp.float32) + sb2_ref[...]
    logits = 0.5 * sem_ref[...] + 0.5 * str_logits

    io = lax.broadcasted_iota(jnp.int32, (BB, NEXP), 1)
    v1 = jnp.max(logits, axis=1, keepdims=True)
    e0 = jnp.min(jnp.where(logits == v1, io, NEXP), axis=1, keepdims=True)
    l2 = jnp.where(io == e0, jnp.float32(-1e30), logits)
    v2 = jnp.max(l2, axis=1, keepdims=True)
    e1 = jnp.min(jnp.where(l2 == v2, io, NEXP), axis=1, keepdims=True)
    sexp = jnp.exp(v2 - v1)
    inv = 1.0 / (1.0 + sexp)
    g0_ref[...] = inv
    g1_ref[...] = sexp * inv
    e0_ref[...] = e0
    e1_ref[...] = e1

    invdeg = 1.0 / jnp.maximum(dc_ref[...], 1.0)
    z = agg_ref[...] * invdeg + h_ref[...]
    sel0 = jnp.zeros((BB, HID), jnp.float32)
    sel1 = jnp.zeros((BB, HID), jnp.float32)
    for e in range(NEXP):
        he = jnp.maximum(
            jnp.dot(z, ew1_ref[e], preferred_element_type=jnp.float32)
            + eb1_ref[e:e + 1, :], 0.0)
        he1_ref[e, :, :] = he
        sel0 += (e0 == e).astype(jnp.float32) * he
        sel1 += (e1 == e).astype(jnp.float32) * he
    sel0_ref[...] = sel0
    sel1_ref[...] = sel1


def _phase_c(bcol, sem, nc, ec32, h, agg0, dcol, str_W1, str_b1,
             str_W2, str_b2, exp_W1, exp_b1):
    full = lambda s: pl.BlockSpec(s, lambda i: tuple(0 for _ in s))
    rowf = pl.BlockSpec((BB, HID), lambda i: (i, 0))
    col = pl.BlockSpec((BB, 1), lambda i: (i, 0))
    return pl.pallas_call(
        _mid_body,
        grid=(NB,),
        in_specs=[
            col,
            pl.BlockSpec((BB, NEXP), lambda i: (i, 0)),
            full((8, NGRAPH)), full((NW, 128)),
            rowf, rowf, col,
            full((2, HID)), full((1, HID)), full((HID, NEXP)),
            full((1, NEXP)), full((NEXP, HID, HID)), full((NEXP, HID)),
        ],
        out_specs=[
            pl.BlockSpec((NEXP, BB, HID), lambda i: (0, i, 0)),
            rowf, rowf, col, col, col, col,
        ],
        out_shape=[
            jax.ShapeDtypeStruct((NEXP, N, HID), jnp.float32),
            jax.ShapeDtypeStruct((N, HID), jnp.float32),
            jax.ShapeDtypeStruct((N, HID), jnp.float32),
            jax.ShapeDtypeStruct((N, 1), jnp.int32),
            jax.ShapeDtypeStruct((N, 1), jnp.int32),
            jax.ShapeDtypeStruct((N, 1), jnp.float32),
            jax.ShapeDtypeStruct((N, 1), jnp.float32),
        ],
    )(bcol, sem, nc, ec32, h, agg0, dcol, str_W1,
      str_b1.reshape(1, HID), str_W2, str_b2.reshape(1, NEXP), exp_W1, exp_b1)


def _phase_d_body(he1_hbm, src_hbm, dst_hbm, ek_hbm, a_out,
                  ek_v, srcv, dstv, cgi, cdst, rows_v, acc_v, sem):
    cid = lax.axis_index("c")
    tid = lax.axis_index("s")
    wid = cid * NS + tid
    base = wid * OWN
    io16 = lax.iota(jnp.int32, 16)

    _zero_acc(acc_v)
    pltpu.sync_copy(ek_hbm, ek_v)

    def _chunk(ci, _):
        off = pl.multiple_of(ci * CH, 8)
        pltpu.sync_copy(src_hbm.at[pl.ds(off, CH)], srcv)
        pltpu.sync_copy(dst_hbm.at[pl.ds(off, CH)], dstv)

        def _pf(r, _):
            cgi[pl.ds(r * 16, 16)] = io16 * 0
            cdst[pl.ds(r * 16, 16)] = OWN + (io16 & 15)
            return 0

        lax.fori_loop(0, CH // 16, _pf, 0)

        lane15 = jnp.full((16,), 15, jnp.int32)

        def _scan(j, mv):
            s = srcv[pl.ds(j * 16, 16)]
            d = dstv[pl.ds(j * 16, 16)]
            dl = d - base
            msk = (dl >= 0) & (dl < OWN)
            csum = plsc.cumsum(msk.astype(jnp.int32))
            pos = mv + csum - 1
            ee = plsc.load_gather(ek_v, [d])
            plsc.store_scatter(cgi, [pos], ee * N + s, mask=msk)
            plsc.store_scatter(cdst, [pos], dl, mask=msk)
            return mv + jnp.take(csum, lane15)

        mv = lax.fori_loop(0, CH // 16, _scan,
                           jnp.zeros((16,), jnp.int32))
        m = mv[0]
        ng = (m + (G - 1)) // G

        def _fire(g, _):
            go = pl.multiple_of(g * G, 8)
            pltpu.async_copy(he1_hbm.at[cgi.at[pl.ds(go, G)]], rows_v,
                             sem).wait()

            def _addgrp(rg, _):
                lbase = go + rg * 16
                slot16 = cdst[pl.ds(lbase, 16)]
                for r in range(16):
                    slot = slot16[r]
                    for c in range(HID // 16):
                        sl = pl.ds(c * 16, 16)
                        plsc.addupdate(acc_v.at[slot, sl],
                                       rows_v[rg * 16 + r, sl])
                return 0

            lax.fori_loop(0, G // 16, _addgrp, 0)
            return 0

        lax.fori_loop(0, ng, _fire, 0)
        return 0

    lax.fori_loop(0, NCHUNK, _chunk, 0)
    _copy_out(acc_v, a_out, wid)


@functools.lru_cache(maxsize=1)
def _phase_d():
    return pl.kernel(
        _phase_d_body, mesh=_mesh(),
        compiler_params=pltpu.CompilerParams(needs_layout_passes=False),
        out_type=jax.ShapeDtypeStruct((NPAD, HID), jnp.float32),
        scratch_types=[
            pltpu.VMEM((N,), jnp.int32),
            pltpu.VMEM((CH,), jnp.int32),
            pltpu.VMEM((CH,), jnp.int32),
            pltpu.VMEM((CH + 16,), jnp.int32),
            pltpu.VMEM((CH + 16,), jnp.int32),
            pltpu.VMEM((G, HID), jnp.float32),
            pltpu.VMEM((ACCR, HID), jnp.float32),
            pltpu.SemaphoreType.DMA,
        ])


def _out_body(a0_ref, a1_ref, sel0_ref, sel1_ref, e0_ref, e1_ref, g0_ref,
              g1_ref, dc_ref, ew2_ref, eb2_ref, out_ref):
    invdeg = 1.0 / jnp.maximum(dc_ref[...], 1.0)
    z0 = a0_ref[...] * invdeg + sel0_ref[...]
    z1 = a1_ref[...] * invdeg + sel1_ref[...]
    e0 = e0_ref[...]
    e1 = e1_ref[...]
    g0 = g0_ref[...]
    g1 = g1_ref[...]
    out = jnp.zeros((BB, HID), jnp.float32)
    for e in range(NEXP):
        y0 = jnp.maximum(
            jnp.dot(z0, ew2_ref[e], preferred_element_type=jnp.float32)
            + eb2_ref[e:e + 1, :], 0.0)
        out += (g0 * (e0 == e).astype(jnp.float32)) * y0
        y1 = jnp.maximum(
            jnp.dot(z1, ew2_ref[e], preferred_element_type=jnp.float32)
            + eb2_ref[e:e + 1, :], 0.0)
        out += (g1 * (e1 == e).astype(jnp.float32)) * y1
    out_ref[...] = out


def _phase_e(a0, a1, sel0, sel1, e0, e1, g0, g1, dcol, exp_W2, exp_b2):
    full = lambda s: pl.BlockSpec(s, lambda i: tuple(0 for _ in s))
    rowf = pl.BlockSpec((BB, HID), lambda i: (i, 0))
    col = pl.BlockSpec((BB, 1), lambda i: (i, 0))
    return pl.pallas_call(
        _out_body,
        grid=(NB,),
        in_specs=[rowf, rowf, rowf, rowf, col, col, col, col, col,
                  full((NEXP, HID, HID)), full((NEXP, HID))],
        out_specs=rowf,
        out_shape=jax.ShapeDtypeStruct((N, HID), jnp.float32),
    )(a0, a1, sel0, sel1, e0, e1, g0, g1, dcol, exp_W2, exp_b2)


def kernel(x, edge_index, batch, W_enc, b_enc, sem_W1, sem_b1, sem_W2, sem_b2,
           str_W1, str_b1, str_W2, str_b2, exp_W1, exp_b1, exp_W2, exp_b2):
    src = edge_index[0]
    dst = edge_index[1]
    bcol = batch.reshape(N, 1)

    h, sem, nc = _phase_a(x, bcol, W_enc, b_enc, sem_W1, sem_b1, sem_W2,
                          sem_b2)

    aggB, dcntB, ecntB = _phase_b()(h, src, dst, batch)
    agg0 = aggB[:N]
    dcol = dcntB[:N, None]

    he1, sel0, sel1, e0c, e1c, g0c, g1c = _phase_c(
        bcol, sem, nc, ecntB, h, agg0, dcol, str_W1, str_b1,
        str_W2, str_b2, exp_W1, exp_b1)

    he1_sc = he1.reshape(NEXP * N, HID)
    a0B = _phase_d()(he1_sc, src, dst, e0c.reshape(N))
    a1B = _phase_d()(he1_sc, src, dst, e1c.reshape(N))

    return _phase_e(a0B[:N], a1B[:N], sel0, sel1, e0c, e1c, g0c, g1c, dcol,
                    exp_W2, exp_b2)
